# baseline (speedup 1.0000x reference)
import functools

import jax
import jax.numpy as jnp
from jax import lax
from jax.experimental import pallas as pl
from jax.experimental.pallas import tpu as pltpu
from jax.experimental.pallas import tpu_sc as plsc

VOCAB = 1000000
D = 64
N = 4096 * 200
NW = 32
PER_W = N // NW
CH = 1024
NCH = PER_W // CH

_mesh = plsc.VectorSubcoreMesh(core_axis_name="c", subcore_axis_name="s")


@functools.partial(
    pl.kernel,
    mesh=_mesh,
    out_type=jax.ShapeDtypeStruct((N, D), jnp.float32),
    scratch_types=[
        pltpu.VMEM((CH,), jnp.int32),
        pltpu.VMEM((CH, D), jnp.float32),
        pltpu.SemaphoreType.DMA,
    ],
    compiler_params=pltpu.CompilerParams(use_tc_tiling_on_sc=False),
)
def _emb_lookup(table_hbm, idx_hbm, out_hbm, idx_v, rows_v, sem):
    wid = lax.axis_index("s") * 2 + lax.axis_index("c")
    base = wid * PER_W

    def body(c, carry):
        off = base + c * CH
        pltpu.sync_copy(idx_hbm.at[pl.ds(off, CH)], idx_v)
        pltpu.async_copy(table_hbm.at[idx_v], rows_v, sem).wait()
        pltpu.sync_copy(rows_v, out_hbm.at[pl.ds(off, CH)])
        return carry

    lax.fori_loop(0, NCH, body, 0)


def kernel(x, table):
    idx = x.reshape(-1).astype(jnp.int32)
    out = _emb_lookup(table, idx)
    return out.reshape(x.shape + (table.shape[-1],))


# speedup vs baseline: 1.0086x; 1.0086x over previous
import functools

import jax
import jax.numpy as jnp
from jax import lax
from jax.experimental import pallas as pl
from jax.experimental.pallas import tpu as pltpu
from jax.experimental.pallas import tpu_sc as plsc

D = 64
N = 4096 * 200
NW = 32
PER_W = N // NW
CH = 800
NCH = PER_W // CH
NBUF = 2

_mesh = plsc.VectorSubcoreMesh(core_axis_name="c", subcore_axis_name="s")


@functools.partial(
    pl.kernel,
    mesh=_mesh,
    out_type=jax.ShapeDtypeStruct((N, D), jnp.float32),
    scratch_types=[
        pltpu.VMEM((NBUF, CH), jnp.int32),
        pltpu.VMEM((NBUF, CH, D), jnp.float32),
        [pltpu.SemaphoreType.DMA] * NBUF,
        [pltpu.SemaphoreType.DMA] * NBUF,
    ],
    compiler_params=pltpu.CompilerParams(use_tc_tiling_on_sc=False),
)
def _emb_lookup(table_hbm, idx_hbm, out_hbm, idx_v, rows_v, gsem, wsem):
    wid = lax.axis_index("s") * 2 + lax.axis_index("c")
    base = wid * PER_W

    gather = [None] * NBUF
    write = [None] * NBUF
    for g in range(NCH):
        b = g % NBUF
        if write[b] is not None:
            write[b].wait()
        pltpu.sync_copy(idx_hbm.at[pl.ds(base + g * CH, CH)], idx_v.at[b])
        gather[b] = pltpu.async_copy(table_hbm.at[idx_v.at[b]], rows_v.at[b],
                                     gsem[b])
        pb = (g - 1) % NBUF
        if g >= 1:
            gather[pb].wait()
            write[pb] = pltpu.async_copy(
                rows_v.at[pb], out_hbm.at[pl.ds(base + (g - 1) * CH, CH)],
                wsem[pb])
    lb = (NCH - 1) % NBUF
    gather[lb].wait()
    write[lb] = pltpu.async_copy(
        rows_v.at[lb], out_hbm.at[pl.ds(base + (NCH - 1) * CH, CH)], wsem[lb])
    for b in range(NBUF):
        if write[b] is not None:
            write[b].wait()


def kernel(x, table):
    idx = x.reshape(-1).astype(jnp.int32)
    out = _emb_lookup(table, idx)
    return out.reshape(x.shape + (table.shape[-1],))


# speedup vs baseline: 1.0175x; 1.0089x over previous
import functools

import jax
import jax.numpy as jnp
from jax import lax
from jax.experimental import pallas as pl
from jax.experimental.pallas import tpu as pltpu
from jax.experimental.pallas import tpu_sc as plsc

D = 64
N = 4096 * 200
NW = 32
PER_W = N // NW
W = 256
NWIN = PER_W // W
NBUF = 4
NG = W // 16

_mesh = plsc.VectorSubcoreMesh(core_axis_name="c", subcore_axis_name="s")


@functools.partial(
    pl.kernel,
    mesh=_mesh,
    out_type=jax.ShapeDtypeStruct((N, D), jnp.float32),
    scratch_types=[
        pltpu.VMEM((PER_W,), jnp.int32),
        pltpu.VMEM((NBUF, W, D), jnp.float32),
        [pltpu.SemaphoreType.DMA] * NBUF,
        [pltpu.SemaphoreType.DMA] * NBUF,
    ],
    compiler_params=pltpu.CompilerParams(use_tc_tiling_on_sc=False),
)
def _emb_lookup(table_hbm, idx_hbm, out_hbm, idx_all, rows, gsem, wsem):
    wid = lax.axis_index("s") * 2 + lax.axis_index("c")
    base = wid * PER_W
    pltpu.sync_copy(idx_hbm.at[pl.ds(base, PER_W)], idx_all)

    def fire(g, b):
        for i in range(NG):
            iv = idx_all[pl.ds(g * W + i * 16, 16)]
            pltpu.async_copy(table_hbm.at[iv], rows.at[b, pl.ds(i * 16, 16)],
                             gsem[b])

    def drain(b):
        for i in range(NG):
            pltpu.make_async_copy(
                table_hbm.at[idx_all[pl.ds(i * 16, 16)]],
                rows.at[b, pl.ds(i * 16, 16)], gsem[b]).wait()

    def start_write(g, b):
        return pltpu.async_copy(
            rows.at[b], out_hbm.at[pl.ds(base + g * W, W)], wsem[b])

    def wait_write(b):
        pltpu.make_async_copy(rows.at[b], out_hbm.at[pl.ds(base, W)],
                              wsem[b]).wait()

    fire(0, 0)
    fire(1, 1)

    def block(m, carry):
        for b in range(NBUF):
            g = m * NBUF + b
            drain(b)
            start_write(g, b)

            @pl.when(g >= 2)
            def _():
                wait_write((b + 2) % NBUF)

            @pl.when(g + 2 < NWIN)
            def _():
                fire(g + 2, (b + 2) % NBUF)
        return carry

    lax.fori_loop(0, NWIN // NBUF, block, 0)
    wait_write((NWIN - 2) % NBUF)
    wait_write((NWIN - 1) % NBUF)


def kernel(x, table):
    idx = x.reshape(-1).astype(jnp.int32)
    out = _emb_lookup(table, idx)
    return out.reshape(x.shape + (table.shape[-1],))
